# baseline (speedup 1.0000x reference)
import jax
import jax.numpy as jnp
from jax import lax
from jax.experimental import pallas as pl
from jax.experimental.pallas import tpu as pltpu

N = 16384
D_IN = 64
D_OUT = 64
BM = 128


def _fused_kernel(xt_ref, w_ref, bias_ref, adj_ref, out_ref, st_ref, bt_ref):
    @pl.when(pl.program_id(0) == 0)
    def _():
        st_ref[...] = lax.dot_general(
            w_ref[...], xt_ref[...],
            dimension_numbers=(((0,), (0,)), ((), ())),
            preferred_element_type=jnp.float32).astype(jnp.bfloat16)
        bt_ref[...] = bias_ref[...].T

    out_ref[...] = lax.dot_general(
        st_ref[...], adj_ref[...],
        dimension_numbers=(((1,), (1,)), ((), ())),
        preferred_element_type=jnp.float32) + bt_ref[...]


@jax.jit
def kernel(x, adj, weight, bias):
    xt = x.T
    bias2d = bias.reshape(1, D_OUT)
    out_t = pl.pallas_call(
        _fused_kernel,
        grid=(N // BM,),
        in_specs=[
            pl.BlockSpec((D_IN, N), lambda i: (0, 0)),
            pl.BlockSpec((D_IN, D_OUT), lambda i: (0, 0)),
            pl.BlockSpec((1, D_OUT), lambda i: (0, 0)),
            pl.BlockSpec((BM, N), lambda i: (i, 0)),
        ],
        out_specs=pl.BlockSpec((D_OUT, BM), lambda i: (0, i)),
        out_shape=jax.ShapeDtypeStruct((D_OUT, N), jnp.float32),
        scratch_shapes=[pltpu.VMEM((D_OUT, N), jnp.bfloat16),
                        pltpu.VMEM((D_OUT, 1), jnp.float32)],
        compiler_params=pltpu.CompilerParams(
            dimension_semantics=("arbitrary",),
        ),
    )(xt, weight, bias2d, adj)
    return out_t.T


# speedup vs baseline: 1.0499x; 1.0499x over previous
import jax
import jax.numpy as jnp
from jax import lax
from jax.experimental import pallas as pl
from jax.experimental.pallas import tpu as pltpu

N = 16384
D_IN = 64
D_OUT = 64
BM = 256


def _fused_kernel(xt_ref, w_ref, bias_ref, adj_ref, out_ref, st_ref, bt_ref):
    @pl.when(pl.program_id(0) == 0)
    def _():
        st_ref[...] = lax.dot_general(
            w_ref[...], xt_ref[...],
            dimension_numbers=(((0,), (0,)), ((), ())),
            preferred_element_type=jnp.float32).astype(jnp.bfloat16)
        bt_ref[...] = bias_ref[...].T

    out_ref[...] = lax.dot_general(
        st_ref[...], adj_ref[...],
        dimension_numbers=(((1,), (1,)), ((), ())),
        preferred_element_type=jnp.float32) + bt_ref[...]


@jax.jit
def kernel(x, adj, weight, bias):
    xt = x.T
    bias2d = bias.reshape(1, D_OUT)
    out_t = pl.pallas_call(
        _fused_kernel,
        grid=(N // BM,),
        in_specs=[
            pl.BlockSpec((D_IN, N), lambda i: (0, 0)),
            pl.BlockSpec((D_IN, D_OUT), lambda i: (0, 0)),
            pl.BlockSpec((1, D_OUT), lambda i: (0, 0)),
            pl.BlockSpec((BM, N), lambda i: (i, 0)),
        ],
        out_specs=pl.BlockSpec((D_OUT, BM), lambda i: (0, i)),
        out_shape=jax.ShapeDtypeStruct((D_OUT, N), jnp.float32),
        scratch_shapes=[pltpu.VMEM((D_OUT, N), jnp.bfloat16),
                        pltpu.VMEM((D_OUT, 1), jnp.float32)],
        compiler_params=pltpu.CompilerParams(
            dimension_semantics=("arbitrary",),
        ),
    )(xt, weight, bias2d, adj)
    return out_t.T
